# Initial kernel scaffold; baseline (speedup 1.0000x reference)
import jax
import jax.numpy as jnp
from jax.experimental import pallas as pl


def kernel(x, ei, W1, b1, g1, be1, W2, b2, g2, be2, Wl1, bl1, Wl2, bl2):
    raise NotImplementedError("write your pallas kernel here")


# speedup vs baseline: 4.1584x; 4.1584x over previous
import functools

import jax
import jax.numpy as jnp
from jax import lax
from jax.experimental import pallas as pl
from jax.experimental.pallas import tpu as pltpu
from jax.experimental.pallas import tpu_sc as plsc

_N = 10000
_E = 320000
_H = 128
_NS = 16
_ROWS_A = 632
_ROWS_LAST = _N - 15 * _ROWS_A
_EDGES_PER_TILE = _E // _NS
_CHUNK = 80
_NCHUNK = _EDGES_PER_TILE // _CHUNK


def _sc_body(x_hbm, ei0_hbm, ei1_hbm, out_hbm, acc, sidx, didx, rows, sem):
    c = lax.axis_index("c")
    s = lax.axis_index("s")

    r0 = s * _ROWS_A

    @pl.when(s < _NS - 1)
    def _():
        pltpu.sync_copy(x_hbm.at[pl.ds(r0, _ROWS_A)],
                        acc.at[pl.ds(r0, _ROWS_A)])

    @pl.when(s == _NS - 1)
    def _():
        pltpu.sync_copy(x_hbm.at[pl.ds(15 * _ROWS_A, _ROWS_LAST)],
                        acc.at[pl.ds(15 * _ROWS_A, _ROWS_LAST)])

    plsc.subcore_barrier()

    ebase = s * _EDGES_PER_TILE

    def run_direction(src_hbm, dst_hbm):
        def chunk_body(j, carry):
            off = ebase + j * _CHUNK
            pltpu.sync_copy(src_hbm.at[pl.ds(off, _CHUNK)], sidx)
            pltpu.sync_copy(dst_hbm.at[pl.ds(off, _CHUNK)], didx)
            pltpu.async_copy(x_hbm.at[sidx], rows, sem).wait()
            pltpu.sync_copy(rows, acc.at[didx], add=True)
            return carry

        lax.fori_loop(0, _NCHUNK, chunk_body, 0)

    @pl.when(c == 0)
    def _():
        run_direction(ei0_hbm, ei1_hbm)

    @pl.when(c == 1)
    def _():
        run_direction(ei1_hbm, ei0_hbm)

    plsc.subcore_barrier()

    @pl.when(s < _NS - 1)
    def _():
        pltpu.sync_copy(acc.at[pl.ds(r0, _ROWS_A)],
                        out_hbm.at[c, pl.ds(r0, _ROWS_A)])

    @pl.when(s == _NS - 1)
    def _():
        pltpu.sync_copy(acc.at[pl.ds(15 * _ROWS_A, _ROWS_LAST)],
                        out_hbm.at[c, pl.ds(15 * _ROWS_A, _ROWS_LAST)])


_sc_agg = functools.partial(
    pl.kernel,
    out_type=jax.ShapeDtypeStruct((2, _N, _H), jnp.float32),
    mesh=plsc.VectorSubcoreMesh(core_axis_name="c", subcore_axis_name="s"),
    scratch_types=[
        pltpu.VMEM_SHARED((_N, _H), jnp.float32),
        pltpu.VMEM((_CHUNK,), jnp.int32),
        pltpu.VMEM((_CHUNK,), jnp.int32),
        pltpu.VMEM((_CHUNK, _H), jnp.float32),
        pltpu.SemaphoreType.DMA,
    ],
)(_sc_body)


_BLK = 1000


def _dense_body(h1_ref, h2_ref, W1t_ref, b1_ref, g1_ref, be1_ref,
                W2t_ref, b2_ref, g2_ref, be2_ref,
                Wl1a_ref, Wl1b_ref, bl1_ref, Wl2t_ref, bl2_ref, out_ref):
    def resblock(h, Wt, b, g, be):
        z = jnp.dot(h, Wt, preferred_element_type=jnp.float32) + b
        mu = jnp.mean(z, axis=-1, keepdims=True)
        var = jnp.mean((z - mu) * (z - mu), axis=-1, keepdims=True)
        ln = (z - mu) * lax.rsqrt(var + 1e-5) * g + be
        return h + jnp.maximum(ln, 0.0)

    r1 = resblock(h1_ref[:], W1t_ref[:], b1_ref[:], g1_ref[:], be1_ref[:])
    r2 = resblock(h2_ref[:], W2t_ref[:], b2_ref[:], g2_ref[:], be2_ref[:])
    hmid = jnp.maximum(
        jnp.dot(r1, Wl1a_ref[:], preferred_element_type=jnp.float32)
        + jnp.dot(r2, Wl1b_ref[:], preferred_element_type=jnp.float32)
        + bl1_ref[:], 0.0)
    out_ref[:] = jnp.maximum(
        jnp.dot(hmid, Wl2t_ref[:], preferred_element_type=jnp.float32)
        + bl2_ref[:], 0.0)


def _row_spec(nrows, ncols):
    return pl.BlockSpec((nrows, ncols), lambda i: (i, 0))


def _full_spec(nrows, ncols):
    return pl.BlockSpec((nrows, ncols), lambda i: (0, 0))


_dense_call = pl.pallas_call(
    _dense_body,
    grid=(_N // _BLK,),
    in_specs=[
        _row_spec(_BLK, _H), _row_spec(_BLK, _H),
        _full_spec(_H, _H), _full_spec(1, _H), _full_spec(1, _H), _full_spec(1, _H),
        _full_spec(_H, _H), _full_spec(1, _H), _full_spec(1, _H), _full_spec(1, _H),
        _full_spec(_H, 2 * _H), _full_spec(_H, 2 * _H), _full_spec(1, 2 * _H),
        _full_spec(2 * _H, _H), _full_spec(1, _H),
    ],
    out_specs=_row_spec(_BLK, _H),
    out_shape=jax.ShapeDtypeStruct((_N, _H), jnp.float32),
)


@jax.jit
def _impl(x, ei, W1, b1, g1, be1, W2, b2, g2, be2, Wl1, bl1, Wl2, bl2):
    h12 = _sc_agg(x, ei[0], ei[1])
    return _dense_call(
        h12[0], h12[1],
        W1.T, b1[None, :], g1[None, :], be1[None, :],
        W2.T, b2[None, :], g2[None, :], be2[None, :],
        Wl1.T[:_H], Wl1.T[_H:], bl1[None, :],
        Wl2.T, bl2[None, :],
    )


def kernel(x, ei, W1, b1, g1, be1, W2, b2, g2, be2, Wl1, bl1, Wl2, bl2):
    return _impl(x, ei, W1, b1, g1, be1, W2, b2, g2, be2, Wl1, bl1, Wl2, bl2)


# speedup vs baseline: 8.0983x; 1.9474x over previous
import functools

import jax
import jax.numpy as jnp
from jax import lax
from jax.experimental import pallas as pl
from jax.experimental.pallas import tpu as pltpu
from jax.experimental.pallas import tpu_sc as plsc

_N = 10000
_E = 320000
_H = 128
_NS = 16
_ROWS_A = 632
_ROWS_LAST = _N - 15 * _ROWS_A
_EDGES_PER_TILE = _E // _NS
_CHUNK = 128
_NFULL = _EDGES_PER_TILE // _CHUNK
_TAIL = _EDGES_PER_TILE - _NFULL * _CHUNK
_NPAIR = _NFULL // 2


def _sc_body(x_hbm, ei0_hbm, ei1_hbm, out_hbm, acc,
             si0, si1, di0, di1, rb0, rb1, sit, dit, rbt,
             isem0, isem1, gsem0, gsem1, tsem):
    c = lax.axis_index("c")
    s = lax.axis_index("s")

    r0 = s * _ROWS_A

    @pl.when(s < _NS - 1)
    def _():
        pltpu.sync_copy(x_hbm.at[pl.ds(r0, _ROWS_A)],
                        acc.at[pl.ds(r0, _ROWS_A)])

    @pl.when(s == _NS - 1)
    def _():
        pltpu.sync_copy(x_hbm.at[pl.ds(15 * _ROWS_A, _ROWS_LAST)],
                        acc.at[pl.ds(15 * _ROWS_A, _ROWS_LAST)])

    plsc.subcore_barrier()

    ebase = s * _EDGES_PER_TILE

    def run_direction(src_hbm, dst_hbm):
        def issue_idx(j, si, di, isem):
            off = ebase + j * _CHUNK
            pltpu.async_copy(src_hbm.at[pl.ds(off, _CHUNK)], si, isem)
            pltpu.async_copy(dst_hbm.at[pl.ds(off, _CHUNK)], di, isem)

        def wait_idx(si, di, isem):
            pltpu.make_async_copy(src_hbm.at[pl.ds(ebase, _CHUNK)], si,
                                  isem).wait()
            pltpu.make_async_copy(dst_hbm.at[pl.ds(ebase, _CHUNK)], di,
                                  isem).wait()

        def wait_gather(rb, gsem):
            pltpu.make_async_copy(x_hbm.at[pl.ds(0, _CHUNK)], rb, gsem).wait()

        issue_idx(0, si0, di0, isem0)
        issue_idx(1, si1, di1, isem1)
        wait_idx(si0, di0, isem0)
        pltpu.async_copy(x_hbm.at[si0], rb0, gsem0)

        def pair_body(k, carry):
            wait_idx(si1, di1, isem1)
            wait_gather(rb0, gsem0)
            pltpu.async_copy(x_hbm.at[si1], rb1, gsem1)
            pltpu.sync_copy(rb0, acc.at[di0], add=True)

            @pl.when(k < _NPAIR - 1)
            def _():
                issue_idx(2 * k + 2, si0, di0, isem0)

            wait_gather(rb1, gsem1)
            pltpu.sync_copy(rb1, acc.at[di1], add=True)

            @pl.when(k < _NPAIR - 1)
            def _():
                wait_idx(si0, di0, isem0)
                pltpu.async_copy(x_hbm.at[si0], rb0, gsem0)
                issue_idx(2 * k + 3, si1, di1, isem1)

            return carry

        lax.fori_loop(0, _NPAIR, pair_body, 0)

        toff = ebase + _NFULL * _CHUNK
        pltpu.sync_copy(src_hbm.at[pl.ds(toff, _TAIL)], sit)
        pltpu.sync_copy(dst_hbm.at[pl.ds(toff, _TAIL)], dit)
        pltpu.async_copy(x_hbm.at[sit], rbt, tsem).wait()
        pltpu.sync_copy(rbt, acc.at[dit], add=True)

    @pl.when(c == 0)
    def _():
        run_direction(ei0_hbm, ei1_hbm)

    @pl.when(c == 1)
    def _():
        run_direction(ei1_hbm, ei0_hbm)

    plsc.subcore_barrier()

    @pl.when(s < _NS - 1)
    def _():
        pltpu.sync_copy(acc.at[pl.ds(r0, _ROWS_A)],
                        out_hbm.at[c, pl.ds(r0, _ROWS_A)])

    @pl.when(s == _NS - 1)
    def _():
        pltpu.sync_copy(acc.at[pl.ds(15 * _ROWS_A, _ROWS_LAST)],
                        out_hbm.at[c, pl.ds(15 * _ROWS_A, _ROWS_LAST)])


_sc_agg = functools.partial(
    pl.kernel,
    out_type=jax.ShapeDtypeStruct((2, _N, _H), jnp.float32),
    mesh=plsc.VectorSubcoreMesh(core_axis_name="c", subcore_axis_name="s"),
    scratch_types=[
        pltpu.VMEM_SHARED((_N, _H), jnp.float32),
        pltpu.VMEM((_CHUNK,), jnp.int32),
        pltpu.VMEM((_CHUNK,), jnp.int32),
        pltpu.VMEM((_CHUNK,), jnp.int32),
        pltpu.VMEM((_CHUNK,), jnp.int32),
        pltpu.VMEM((_CHUNK, _H), jnp.float32),
        pltpu.VMEM((_CHUNK, _H), jnp.float32),
        pltpu.VMEM((_TAIL,), jnp.int32),
        pltpu.VMEM((_TAIL,), jnp.int32),
        pltpu.VMEM((_TAIL, _H), jnp.float32),
        pltpu.SemaphoreType.DMA,
        pltpu.SemaphoreType.DMA,
        pltpu.SemaphoreType.DMA,
        pltpu.SemaphoreType.DMA,
        pltpu.SemaphoreType.DMA,
    ],
)(_sc_body)


_BLK = 1000


def _dense_body(h1_ref, h2_ref, W1t_ref, b1_ref, g1_ref, be1_ref,
                W2t_ref, b2_ref, g2_ref, be2_ref,
                Wl1a_ref, Wl1b_ref, bl1_ref, Wl2t_ref, bl2_ref, out_ref):
    def resblock(h, Wt, b, g, be):
        z = jnp.dot(h, Wt, preferred_element_type=jnp.float32) + b
        mu = jnp.mean(z, axis=-1, keepdims=True)
        var = jnp.mean((z - mu) * (z - mu), axis=-1, keepdims=True)
        ln = (z - mu) * lax.rsqrt(var + 1e-5) * g + be
        return h + jnp.maximum(ln, 0.0)

    r1 = resblock(h1_ref[:], W1t_ref[:], b1_ref[:], g1_ref[:], be1_ref[:])
    r2 = resblock(h2_ref[:], W2t_ref[:], b2_ref[:], g2_ref[:], be2_ref[:])
    hmid = jnp.maximum(
        jnp.dot(r1, Wl1a_ref[:], preferred_element_type=jnp.float32)
        + jnp.dot(r2, Wl1b_ref[:], preferred_element_type=jnp.float32)
        + bl1_ref[:], 0.0)
    out_ref[:] = jnp.maximum(
        jnp.dot(hmid, Wl2t_ref[:], preferred_element_type=jnp.float32)
        + bl2_ref[:], 0.0)


def _row_spec(nrows, ncols):
    return pl.BlockSpec((nrows, ncols), lambda i: (i, 0))


def _full_spec(nrows, ncols):
    return pl.BlockSpec((nrows, ncols), lambda i: (0, 0))


_dense_call = pl.pallas_call(
    _dense_body,
    grid=(_N // _BLK,),
    in_specs=[
        _row_spec(_BLK, _H), _row_spec(_BLK, _H),
        _full_spec(_H, _H), _full_spec(1, _H), _full_spec(1, _H), _full_spec(1, _H),
        _full_spec(_H, _H), _full_spec(1, _H), _full_spec(1, _H), _full_spec(1, _H),
        _full_spec(_H, 2 * _H), _full_spec(_H, 2 * _H), _full_spec(1, 2 * _H),
        _full_spec(2 * _H, _H), _full_spec(1, _H),
    ],
    out_specs=_row_spec(_BLK, _H),
    out_shape=jax.ShapeDtypeStruct((_N, _H), jnp.float32),
)


@jax.jit
def _impl(x, ei, W1, b1, g1, be1, W2, b2, g2, be2, Wl1, bl1, Wl2, bl2):
    h12 = _sc_agg(x, ei[0], ei[1])
    return _dense_call(
        h12[0], h12[1],
        W1.T, b1[None, :], g1[None, :], be1[None, :],
        W2.T, b2[None, :], g2[None, :], be2[None, :],
        Wl1.T[:_H], Wl1.T[_H:], bl1[None, :],
        Wl2.T, bl2[None, :],
    )


def kernel(x, ei, W1, b1, g1, be1, W2, b2, g2, be2, Wl1, bl1, Wl2, bl2):
    return _impl(x, ei, W1, b1, g1, be1, W2, b2, g2, be2, Wl1, bl1, Wl2, bl2)
